# baseline (speedup 1.0000x reference)
import functools
import math

import jax
import jax.numpy as jnp
from jax import lax
from jax.experimental import pallas as pl
from jax.experimental.pallas import tpu as pltpu
from jax.experimental.pallas import tpu_sc as plsc

VOCAB = 1000000
DIM = 64
MAX_LEN = 200
BATCH = 1024
SEQ = 200

NC = 2
NS = 16
NW = NC * NS

ROWS = BATCH * SEQ
ROWS_PER_W = ROWS // NW
CHUNK = 100
NCHUNK = ROWS_PER_W // CHUNK
NG = 8
NO = 4
LANES = 16
SCALE = math.sqrt(DIM)


def _make_pe():
    w = jnp.exp(-jnp.arange(0, DIM, 2, dtype=jnp.float32) * math.log(10000.0) / DIM)
    p = jnp.arange(0, MAX_LEN, dtype=jnp.float32).reshape(MAX_LEN, 1)
    pe = jnp.zeros((MAX_LEN, DIM), dtype=jnp.float32)
    pe = pe.at[:, 0::2].set(jnp.sin(p * w))
    pe = pe.at[:, 1::2].set(jnp.cos(p * w))
    return pe


@functools.partial(
    pl.kernel,
    mesh=plsc.VectorSubcoreMesh(core_axis_name="c", subcore_axis_name="s"),
    out_type=jax.ShapeDtypeStruct((ROWS, DIM), jnp.float32),
    scratch_types=[
        pltpu.VMEM((NCHUNK, CHUNK), jnp.int32),
        pltpu.VMEM((MAX_LEN, DIM), jnp.float32),
        pltpu.VMEM((NG, CHUNK, DIM), jnp.float32),
        pltpu.VMEM((NO, CHUNK, DIM), jnp.float32),
        pltpu.SemaphoreType.DMA((NG,)),
        pltpu.SemaphoreType.DMA((NO,)),
    ],
    compiler_params=pltpu.CompilerParams(use_tc_tiling_on_sc=False),
)
def _sc_embed(idx_hbm, pe_hbm, table_hbm, out_hbm, idx_v, pe_v, gbuf, obuf,
              gsem, osem):
    wid = lax.axis_index("s") * NC + lax.axis_index("c")
    base = wid * ROWS_PER_W
    pltpu.sync_copy(idx_hbm.at[pl.ds(wid * NCHUNK, NCHUNK)], idx_v)
    pltpu.sync_copy(pe_hbm, pe_v)

    def issue_gather(c, b):
        pltpu.async_copy(
            table_hbm.at[idx_v.at[c]], gbuf.at[b], gsem.at[b])

    for b in range(NG):
        issue_gather(b, b)

    def group_body(g, carry):
        for b in range(NG):
            c = g * NG + b
            o = b % NO
            pltpu.make_async_copy(
                table_hbm.at[idx_v.at[c]], gbuf.at[b], gsem.at[b]).wait()
            @pl.when(c >= NO)
            def _():
                pltpu.make_async_copy(
                    obuf.at[o], out_hbm.at[pl.ds(0, CHUNK)], osem.at[o]).wait()

            poff = lax.rem(c, 2) * CHUNK

            def row_body(i, rcarry):
                for v in range(DIM // LANES):
                    sl = pl.ds(v * LANES, LANES)
                    obuf[o, i, sl] = gbuf[b, i, sl] * SCALE + pe_v[poff + i, sl]
                return rcarry

            lax.fori_loop(0, CHUNK, row_body, 0)
            pltpu.async_copy(
                obuf.at[o], out_hbm.at[pl.ds(base + c * CHUNK, CHUNK)],
                osem.at[o])

            @pl.when(g < NCHUNK // NG - 1)
            def _():
                issue_gather(c + NG, b)
        return carry

    lax.fori_loop(0, NCHUNK // NG, group_body, 0)

    for o in range(NO):
        pltpu.make_async_copy(
            obuf.at[o], out_hbm.at[pl.ds(0, CHUNK)], osem.at[o]).wait()


def kernel(x, table):
    pe = _make_pe()
    idx = x.reshape(NW * NCHUNK, CHUNK).astype(jnp.int32)
    out = _sc_embed(idx, pe, table)
    return out.reshape(BATCH, SEQ, DIM)
